# initial kernel scaffold (unmeasured)
import jax
import jax.numpy as jnp
from jax import lax
from jax.experimental import pallas as pl
from jax.experimental.pallas import tpu as pltpu


def _exchange_add(partial):
    t, d = partial.shape

    def body(p_ref, out_ref, recv_buf, send_sem, recv_sem):
        my_x = lax.axis_index("x")
        my_y = lax.axis_index("y")
        my_z = lax.axis_index("z")
        nbr = (1 - my_x, my_y, my_z)

        barrier_sem = pltpu.get_barrier_semaphore()
        pl.semaphore_signal(
            barrier_sem, inc=1, device_id=nbr,
            device_id_type=pl.DeviceIdType.MESH,
        )
        pl.semaphore_wait(barrier_sem, 1)

        rdma = pltpu.make_async_remote_copy(
            src_ref=p_ref,
            dst_ref=recv_buf,
            send_sem=send_sem,
            recv_sem=recv_sem,
            device_id=nbr,
            device_id_type=pl.DeviceIdType.MESH,
        )
        rdma.start()
        rdma.wait()

        out_ref[...] = p_ref[...].astype(jnp.float32) + recv_buf[...].astype(
            jnp.float32
        )

    return pl.pallas_call(
        body,
        out_shape=jax.ShapeDtypeStruct((t, d), jnp.float32),
        in_specs=[pl.BlockSpec(memory_space=pltpu.VMEM)],
        out_specs=pl.BlockSpec(memory_space=pltpu.VMEM),
        scratch_shapes=[
            pltpu.VMEM((t, d), jnp.bfloat16),
            pltpu.SemaphoreType.DMA,
            pltpu.SemaphoreType.DMA,
        ],
        compiler_params=pltpu.CompilerParams(collective_id=0),
    )(partial)


def kernel(ids, E):
    v_shard = E.shape[0]
    my_x = lax.axis_index("x")
    local = ids - my_x * v_shard
    mask = (local >= 0) & (local < v_shard)
    rows = E[jnp.clip(local, 0, v_shard - 1)]
    partial = jnp.where(mask[:, None], rows, 0.0).astype(jnp.bfloat16)
    return _exchange_add(partial)


# baseline (device time: 14823 ns/iter reference)
import functools

import jax
import jax.numpy as jnp
from jax import lax
from jax.experimental import pallas as pl
from jax.experimental.pallas import tpu as pltpu


def _exchange_add(partial):
    t, d = partial.shape

    def body(p_ref, out_ref, recv_buf, send_sem, recv_sem):
        my_x = lax.axis_index("x")
        my_y = lax.axis_index("y")
        my_z = lax.axis_index("z")
        nbr = (1 - my_x, my_y, my_z)

        barrier_sem = pltpu.get_barrier_semaphore()
        pl.semaphore_signal(
            barrier_sem, inc=1, device_id=nbr,
            device_id_type=pl.DeviceIdType.MESH,
        )
        pl.semaphore_wait(barrier_sem, 1)

        rdma = pltpu.make_async_remote_copy(
            src_ref=p_ref,
            dst_ref=recv_buf,
            send_sem=send_sem,
            recv_sem=recv_sem,
            device_id=nbr,
            device_id_type=pl.DeviceIdType.MESH,
        )
        rdma.start()
        rdma.wait()

        @functools.partial(pl.run_scoped, ack_sem=pltpu.SemaphoreType.REGULAR)
        def _(ack_sem):
            pl.semaphore_signal(
                ack_sem, inc=1, device_id=nbr,
                device_id_type=pl.DeviceIdType.MESH,
            )
            pl.semaphore_wait(ack_sem, 1)

        out_ref[...] = p_ref[...].astype(jnp.float32) + recv_buf[...].astype(
            jnp.float32
        )

    return pl.pallas_call(
        body,
        out_shape=jax.ShapeDtypeStruct((t, d), jnp.float32),
        in_specs=[pl.BlockSpec(memory_space=pltpu.VMEM)],
        out_specs=pl.BlockSpec(memory_space=pltpu.VMEM),
        scratch_shapes=[
            pltpu.VMEM((t, d), jnp.bfloat16),
            pltpu.SemaphoreType.DMA,
            pltpu.SemaphoreType.DMA,
        ],
        compiler_params=pltpu.CompilerParams(collective_id=0),
    )(partial)


def kernel(ids, E):
    v_shard = E.shape[0]
    my_x = lax.axis_index("x")
    local = ids - my_x * v_shard
    mask = (local >= 0) & (local < v_shard)
    rows = E[jnp.clip(local, 0, v_shard - 1)]
    partial = jnp.where(mask[:, None], rows, 0.0).astype(jnp.bfloat16)
    return _exchange_add(partial)


# device time: 14582 ns/iter; 1.0165x vs baseline; 1.0165x over previous
import functools

import jax
import jax.numpy as jnp
from jax import lax
from jax.experimental import pallas as pl
from jax.experimental.pallas import tpu as pltpu


def _exchange_add(raw, local_ids_2d, v_shard):
    t, d = raw.shape

    def body(
        raw_ref, loc_ref, out_ref,
        send_q, send_s, recv_q, recv_s,
        sem_qs, sem_qr, sem_ss, sem_sr,
    ):
        my_x = lax.axis_index("x")
        my_y = lax.axis_index("y")
        my_z = lax.axis_index("z")
        nbr = (1 - my_x, my_y, my_z)

        loc = loc_ref[...]
        mask = (loc >= 0) & (loc < v_shard)
        own = jnp.where(mask, raw_ref[...], 0.0)
        absmax = jnp.max(jnp.abs(own), axis=1, keepdims=True)
        scale = jnp.maximum(absmax, 1e-30) / 127.0
        send_s[...] = scale
        send_q[...] = jnp.round(own / scale).astype(jnp.int8)

        barrier_sem = pltpu.get_barrier_semaphore()
        pl.semaphore_signal(
            barrier_sem, inc=1, device_id=nbr,
            device_id_type=pl.DeviceIdType.MESH,
        )
        pl.semaphore_wait(barrier_sem, 1)

        rdma_q = pltpu.make_async_remote_copy(
            src_ref=send_q, dst_ref=recv_q,
            send_sem=sem_qs, recv_sem=sem_qr,
            device_id=nbr, device_id_type=pl.DeviceIdType.MESH,
        )
        rdma_s = pltpu.make_async_remote_copy(
            src_ref=send_s, dst_ref=recv_s,
            send_sem=sem_ss, recv_sem=sem_sr,
            device_id=nbr, device_id_type=pl.DeviceIdType.MESH,
        )
        rdma_q.start()
        rdma_s.start()
        rdma_q.wait()
        rdma_s.wait()

        @functools.partial(pl.run_scoped, ack_sem=pltpu.SemaphoreType.REGULAR)
        def _(ack_sem):
            pl.semaphore_signal(
                ack_sem, inc=1, device_id=nbr,
                device_id_type=pl.DeviceIdType.MESH,
            )
            pl.semaphore_wait(ack_sem, 1)

        loc2 = loc_ref[...]
        mask2 = (loc2 >= 0) & (loc2 < v_shard)
        out_ref[...] = (
            jnp.where(mask2, raw_ref[...], 0.0)
            + recv_q[...].astype(jnp.float32) * recv_s[...]
        )

    return pl.pallas_call(
        body,
        out_shape=jax.ShapeDtypeStruct((t, d), jnp.float32),
        in_specs=[
            pl.BlockSpec(memory_space=pltpu.VMEM),
            pl.BlockSpec(memory_space=pltpu.VMEM),
        ],
        out_specs=pl.BlockSpec(memory_space=pltpu.VMEM),
        scratch_shapes=[
            pltpu.VMEM((t, d), jnp.int8),
            pltpu.VMEM((t, 1), jnp.float32),
            pltpu.VMEM((t, d), jnp.int8),
            pltpu.VMEM((t, 1), jnp.float32),
            pltpu.SemaphoreType.DMA,
            pltpu.SemaphoreType.DMA,
            pltpu.SemaphoreType.DMA,
            pltpu.SemaphoreType.DMA,
        ],
        compiler_params=pltpu.CompilerParams(collective_id=0),
    )(raw, local_ids_2d)


def kernel(ids, E):
    v_shard = E.shape[0]
    my_x = lax.axis_index("x")
    local = ids - my_x * v_shard
    raw = E[jnp.clip(local, 0, v_shard - 1)]
    return _exchange_add(raw, local[:, None], v_shard)


# device time: 12832 ns/iter; 1.1552x vs baseline; 1.1364x over previous
import functools

import jax
import jax.numpy as jnp
from jax import lax
from jax.experimental import pallas as pl
from jax.experimental.pallas import tpu as pltpu


def _exchange_add(raw, local_ids_2d, v_shard):
    t, d = raw.shape

    def body(raw_ref, loc_ref, out_ref, send_buf, recv_buf, send_sem, recv_sem):
        my_x = lax.axis_index("x")
        my_y = lax.axis_index("y")
        my_z = lax.axis_index("z")
        nbr = (1 - my_x, my_y, my_z)

        loc = loc_ref[...]
        mask = (loc >= 0) & (loc < v_shard)
        send_buf[...] = jnp.where(mask, raw_ref[...], 0.0).astype(jnp.bfloat16)

        barrier_sem = pltpu.get_barrier_semaphore()
        pl.semaphore_signal(
            barrier_sem, inc=1, device_id=nbr,
            device_id_type=pl.DeviceIdType.MESH,
        )
        pl.semaphore_wait(barrier_sem, 1)

        rdma = pltpu.make_async_remote_copy(
            src_ref=send_buf,
            dst_ref=recv_buf,
            send_sem=send_sem,
            recv_sem=recv_sem,
            device_id=nbr,
            device_id_type=pl.DeviceIdType.MESH,
        )
        rdma.start()
        rdma.wait()

        @functools.partial(pl.run_scoped, ack_sem=pltpu.SemaphoreType.REGULAR)
        def _(ack_sem):
            pl.semaphore_signal(
                ack_sem, inc=1, device_id=nbr,
                device_id_type=pl.DeviceIdType.MESH,
            )
            pl.semaphore_wait(ack_sem, 1)

        out_ref[...] = send_buf[...].astype(jnp.float32) + recv_buf[...].astype(
            jnp.float32
        )

    return pl.pallas_call(
        body,
        out_shape=jax.ShapeDtypeStruct((t, d), jnp.float32),
        in_specs=[
            pl.BlockSpec(memory_space=pltpu.VMEM),
            pl.BlockSpec(memory_space=pltpu.VMEM),
        ],
        out_specs=pl.BlockSpec(memory_space=pltpu.VMEM),
        scratch_shapes=[
            pltpu.VMEM((t, d), jnp.bfloat16),
            pltpu.VMEM((t, d), jnp.bfloat16),
            pltpu.SemaphoreType.DMA,
            pltpu.SemaphoreType.DMA,
        ],
        compiler_params=pltpu.CompilerParams(collective_id=0),
    )(raw, local_ids_2d)


def kernel(ids, E):
    v_shard = E.shape[0]
    my_x = lax.axis_index("x")
    local = ids - my_x * v_shard
    raw = E[jnp.clip(local, 0, v_shard - 1)]
    return _exchange_add(raw, local[:, None], v_shard)
